# Initial kernel scaffold; baseline (speedup 1.0000x reference)
import jax
import jax.numpy as jnp
from jax.experimental import pallas as pl


def kernel(xyz, base_xyz, W_q, b_q, W_k, b_k, W_v, b_v, W_res, b_res, g_res, be_res, W_ffn, b_ffn, g_ffn, be_ffn):
    raise NotImplementedError("write your pallas kernel here")


# speedup vs baseline: 1.4618x; 1.4618x over previous
import jax
import jax.numpy as jnp
import numpy as np
from jax.experimental import pallas as pl
from jax.experimental.pallas import tpu as pltpu

KNN = 32
C_OUT = 128
BLK = 256


def _topk_kernel(x_ref, bt_ref, idx_ref):
    x = x_ref[0]
    bt = bt_ref[0]
    n = bt.shape[1]
    xb = None
    x2 = None
    b2 = None
    for c in range(3):
        xc = x[:, c:c + 1]
        bc = bt[c:c + 1, :]
        xcl = xc.astype(jnp.bfloat16).astype(jnp.float32)
        bcl = bc.astype(jnp.bfloat16).astype(jnp.float32)
        t = xcl * bcl
        xb = t if xb is None else xb + t
        x2 = xc * xc if x2 is None else x2 + xc * xc
        b2 = bc * bc if b2 is None else b2 + bc * bc
    d = ((-2.0 * xb) + x2) + b2
    iota = jax.lax.broadcasted_iota(jnp.int32, d.shape, 1)
    cols = []
    for k in range(KNN):
        rmin = jnp.min(d, axis=1, keepdims=True)
        cand = jnp.where(d == rmin, iota, jnp.int32(n))
        amin = jnp.min(cand, axis=1, keepdims=True)
        cols.append(amin)
        d = jnp.where(iota == amin, jnp.float32(jnp.inf), d)
    idx_ref[0] = jnp.concatenate(cols, axis=1)


def _attn_kernel(x_ref, g_ref, wq_ref, bq_ref, wk_ref, bk_ref, wv_ref,
                 bv_ref, wr_ref, br_ref, wf_ref, bf_ref, yr_ref, yf_ref):
    def _r(v):
        return v.astype(jnp.bfloat16).astype(jnp.float32)

    x = x_ref[...]
    g = g_ref[...]
    wq = _r(wq_ref[...])
    wk = _r(wk_ref[...])
    wv = _r(wv_ref[...])
    wr = _r(wr_ref[...])
    lq = None
    yr = None
    lk = None
    lv = None
    for c in range(3):
        xc = x[:, c:c + 1]
        xcl = _r(xc)
        lq = (0.0 if lq is None else lq) + xcl * wq[c:c + 1, :]
        yr = (0.0 if yr is None else yr) + xcl * wr[c:c + 1, :]
        rel_c = _r(g[:, :, c:c + 1] - xc[:, :, None])
        lk = (0.0 if lk is None else lk) + rel_c * wk[c:c + 1, :][None]
        lv = (0.0 if lv is None else lv) + rel_c * wv[c:c + 1, :][None]
    lq = lq + bq_ref[...]
    yr = yr + br_ref[...]
    lk = lk + bk_ref[...][None]
    lv = lv + bv_ref[...][None]
    energy = (lq[:, None, :] - lk) / np.float32(np.sqrt(C_OUT))
    m = jnp.max(energy, axis=1, keepdims=True)
    e = jnp.exp(energy - m)
    p = e / jnp.sum(e, axis=1, keepdims=True)
    a = p - jnp.sum(p, axis=1, keepdims=True)
    ctx = jnp.max(a * lv, axis=1)
    yr_ref[...] = yr
    yf_ref[...] = jnp.dot(ctx.astype(jnp.bfloat16),
                          wf_ref[...].astype(jnp.bfloat16),
                          preferred_element_type=jnp.float32) + bf_ref[...]


def _leaky(x):
    return jnp.where(x >= 0, x, 0.2 * x)


def _bn_kernel(yr_ref, yf_ref, gr_ref, ber_ref, gf_ref, bef_ref, out_ref):
    def bn(y, gamma, beta):
        mean = jnp.mean(y, axis=0, keepdims=True)
        var = jnp.mean((y - mean) ** 2, axis=0, keepdims=True)
        return gamma * (y - mean) / jnp.sqrt(var + 1e-5) + beta

    hr = _leaky(bn(yr_ref[...], gr_ref[...], ber_ref[...]))
    hf = _leaky(bn(yf_ref[...], gf_ref[...], bef_ref[...]))
    out_ref[...] = hr + hf


def _full(shape):
    nd = len(shape)
    return pl.BlockSpec(shape, lambda *args: (0,) * nd)


def kernel(xyz, base_xyz, W_q, b_q, W_k, b_k, W_v, b_v, W_res, b_res,
           g_res, be_res, W_ffn, b_ffn, g_ffn, be_ffn):
    B, N, _ = xyz.shape
    BN = B * N
    base_t = jnp.transpose(base_xyz, (0, 2, 1))

    idx = pl.pallas_call(
        _topk_kernel,
        grid=(B, N // BLK),
        in_specs=[
            pl.BlockSpec((1, BLK, 3), lambda b, i: (b, i, 0)),
            pl.BlockSpec((1, 3, N), lambda b, i: (b, 0, 0)),
        ],
        out_specs=pl.BlockSpec((1, BLK, KNN), lambda b, i: (b, i, 0)),
        out_shape=jax.ShapeDtypeStruct((B, N, KNN), jnp.int32),
    )(xyz, base_t)

    gath = jax.vmap(lambda p, i: p[i])(xyz, idx)
    GW = 3
    gath_flat = gath.reshape(BN, KNN, GW)

    xyz_flat = xyz.reshape(BN, 3)
    b2 = lambda v: v.reshape(1, C_OUT)
    yr, yf = pl.pallas_call(
        _attn_kernel,
        grid=(BN // BLK,),
        in_specs=[
            pl.BlockSpec((BLK, 3), lambda i: (i, 0)),
            pl.BlockSpec((BLK, KNN, GW), lambda i: (i, 0, 0)),
            _full((3, C_OUT)), _full((1, C_OUT)),
            _full((3, C_OUT)), _full((1, C_OUT)),
            _full((3, C_OUT)), _full((1, C_OUT)),
            _full((3, C_OUT)), _full((1, C_OUT)),
            _full((C_OUT, C_OUT)), _full((1, C_OUT)),
        ],
        out_specs=[
            pl.BlockSpec((BLK, C_OUT), lambda i: (i, 0)),
            pl.BlockSpec((BLK, C_OUT), lambda i: (i, 0)),
        ],
        out_shape=[
            jax.ShapeDtypeStruct((BN, C_OUT), jnp.float32),
            jax.ShapeDtypeStruct((BN, C_OUT), jnp.float32),
        ],
    )(xyz_flat, gath_flat, W_q, b2(b_q), W_k, b2(b_k), W_v, b2(b_v),
      W_res, b2(b_res), W_ffn, b2(b_ffn))

    out = pl.pallas_call(
        _bn_kernel,
        in_specs=[
            _full((BN, C_OUT)), _full((BN, C_OUT)),
            _full((1, C_OUT)), _full((1, C_OUT)),
            _full((1, C_OUT)), _full((1, C_OUT)),
        ],
        out_specs=_full((BN, C_OUT)),
        out_shape=jax.ShapeDtypeStruct((BN, C_OUT), jnp.float32),
    )(yr, yf, b2(g_res), b2(be_res), b2(g_ffn), b2(be_ffn))

    return out.reshape(B, N, C_OUT)


# speedup vs baseline: 8.4610x; 5.7883x over previous
import functools

import jax
import jax.numpy as jnp
import numpy as np
from jax import lax
from jax.experimental import pallas as pl
from jax.experimental.pallas import tpu as pltpu
from jax.experimental.pallas import tpu_sc as plsc

KNN = 32
C_OUT = 128
BLK = 256
GW = 16
CHUNK = 128


def _topk_kernel(x_ref, bt_ref, idx_ref):
    x = x_ref[0]
    bt = bt_ref[0]
    n = bt.shape[1]
    xb = None
    x2 = None
    b2 = None
    for c in range(3):
        xc = x[:, c:c + 1]
        bc = bt[c:c + 1, :]
        xcl = xc.astype(jnp.bfloat16).astype(jnp.float32)
        bcl = bc.astype(jnp.bfloat16).astype(jnp.float32)
        t = xcl * bcl
        xb = t if xb is None else xb + t
        x2 = xc * xc if x2 is None else x2 + xc * xc
        b2 = bc * bc if b2 is None else b2 + bc * bc
    d = ((-2.0 * xb) + x2) + b2
    iota = jax.lax.broadcasted_iota(jnp.int32, d.shape, 1)
    cols = []
    for k in range(KNN):
        rmin = jnp.min(d, axis=1, keepdims=True)
        cand = jnp.where(d == rmin, iota, jnp.int32(n))
        amin = jnp.min(cand, axis=1, keepdims=True)
        cols.append(amin)
        d = jnp.where(iota == amin, jnp.float32(jnp.inf), d)
    base = pl.program_id(0) * n
    idx_ref[0] = jnp.concatenate(cols, axis=1) + base


def _attn_kernel(x_ref, g_ref, wq_ref, bq_ref, wk_ref, bk_ref, wv_ref,
                 bv_ref, wr_ref, br_ref, wf_ref, bf_ref, yr_ref, yf_ref):
    def _r(v):
        return v.astype(jnp.bfloat16).astype(jnp.float32)

    x = x_ref[...]
    g = g_ref[...]
    wq = _r(wq_ref[...])
    wk = _r(wk_ref[...])
    wv = _r(wv_ref[...])
    wr = _r(wr_ref[...])
    lq = None
    yr = None
    lk = None
    lv = None
    for c in range(3):
        xc = x[:, c:c + 1]
        xcl = _r(xc)
        lq = (0.0 if lq is None else lq) + xcl * wq[c:c + 1, :]
        yr = (0.0 if yr is None else yr) + xcl * wr[c:c + 1, :]
        rel_c = _r(g[:, :, c:c + 1] - xc[:, :, None])
        lk = (0.0 if lk is None else lk) + rel_c * wk[c:c + 1, :][None]
        lv = (0.0 if lv is None else lv) + rel_c * wv[c:c + 1, :][None]
    lq = lq + bq_ref[...]
    yr = yr + br_ref[...]
    lk = lk + bk_ref[...][None]
    lv = lv + bv_ref[...][None]
    energy = (lq[:, None, :] - lk) / np.float32(np.sqrt(C_OUT))
    m = jnp.max(energy, axis=1, keepdims=True)
    e = jnp.exp(energy - m)
    p = e / jnp.sum(e, axis=1, keepdims=True)
    a = p - jnp.sum(p, axis=1, keepdims=True)
    ctx = jnp.max(a * lv, axis=1)
    yr_ref[...] = yr
    yf_ref[...] = jnp.dot(ctx.astype(jnp.bfloat16),
                          wf_ref[...].astype(jnp.bfloat16),
                          preferred_element_type=jnp.float32) + bf_ref[...]


def _make_sc_gather(n_idx, n_rows):
    info = plsc.get_sparse_core_info()
    nw = info.num_cores * info.num_subcores
    n_chunks = n_idx // CHUNK
    cpw = n_chunks // nw
    assert cpw % 2 == 0 and n_chunks % nw == 0
    mesh = plsc.VectorSubcoreMesh(core_axis_name="c", subcore_axis_name="s")

    @functools.partial(
        pl.kernel,
        mesh=mesh,
        compiler_params=pltpu.CompilerParams(use_tc_tiling_on_sc=False),
        out_type=jax.ShapeDtypeStruct((n_idx, GW), jnp.float32),
        scratch_types=[
            pltpu.VMEM((cpw, CHUNK), jnp.int32),
            pltpu.VMEM((CHUNK, GW), jnp.float32),
            pltpu.VMEM((CHUNK, GW), jnp.float32),
            pltpu.SemaphoreType.DMA,
            pltpu.SemaphoreType.DMA,
        ],
    )
    def gather(idx_hbm, table_hbm, out_hbm, idx_v, buf_a, buf_b, sem_a, sem_b):
        wid = lax.axis_index("s") * info.num_cores + lax.axis_index("c")
        c0 = wid * cpw
        pltpu.sync_copy(idx_hbm.at[pl.ds(c0, cpw)], idx_v)

        def body(t, _):
            a = 2 * t
            cp_a = pltpu.async_copy(table_hbm.at[idx_v.at[a]], buf_a, sem_a)
            cp_b = pltpu.async_copy(table_hbm.at[idx_v.at[a + 1]], buf_b, sem_b)
            cp_a.wait()
            pltpu.sync_copy(buf_a, out_hbm.at[pl.ds((c0 + a) * CHUNK, CHUNK)])
            cp_b.wait()
            pltpu.sync_copy(buf_b,
                            out_hbm.at[pl.ds((c0 + a + 1) * CHUNK, CHUNK)])
            return 0

        lax.fori_loop(0, cpw // 2, body, 0)

    return gather


def _leaky(x):
    return jnp.where(x >= 0, x, 0.2 * x)


def _bn_kernel(yr_ref, yf_ref, gr_ref, ber_ref, gf_ref, bef_ref, out_ref):
    def bn(y, gamma, beta):
        mean = jnp.mean(y, axis=0, keepdims=True)
        var = jnp.mean((y - mean) ** 2, axis=0, keepdims=True)
        return gamma * (y - mean) / jnp.sqrt(var + 1e-5) + beta

    hr = _leaky(bn(yr_ref[...], gr_ref[...], ber_ref[...]))
    hf = _leaky(bn(yf_ref[...], gf_ref[...], bef_ref[...]))
    out_ref[...] = hr + hf


def _full(shape):
    nd = len(shape)
    return pl.BlockSpec(shape, lambda *args: (0,) * nd)


def kernel(xyz, base_xyz, W_q, b_q, W_k, b_k, W_v, b_v, W_res, b_res,
           g_res, be_res, W_ffn, b_ffn, g_ffn, be_ffn):
    B, N, _ = xyz.shape
    BN = B * N
    base_t = jnp.transpose(base_xyz, (0, 2, 1))

    idx = pl.pallas_call(
        _topk_kernel,
        grid=(B, N // BLK),
        in_specs=[
            pl.BlockSpec((1, BLK, 3), lambda b, i: (b, i, 0)),
            pl.BlockSpec((1, 3, N), lambda b, i: (b, 0, 0)),
        ],
        out_specs=pl.BlockSpec((1, BLK, KNN), lambda b, i: (b, i, 0)),
        out_shape=jax.ShapeDtypeStruct((B, N, KNN), jnp.int32),
    )(xyz, base_t)

    n_idx = BN * KNN
    table = jnp.pad(xyz.reshape(BN, 3), ((0, 0), (0, GW - 3)))
    idx_chunks = idx.reshape(n_idx // CHUNK, CHUNK)
    gath_flat = _make_sc_gather(n_idx, BN)(idx_chunks, table)
    gath_flat = gath_flat.reshape(BN, KNN, GW)

    xyz_flat = xyz.reshape(BN, 3)
    b2 = lambda v: v.reshape(1, C_OUT)
    yr, yf = pl.pallas_call(
        _attn_kernel,
        grid=(BN // BLK,),
        in_specs=[
            pl.BlockSpec((BLK, 3), lambda i: (i, 0)),
            pl.BlockSpec((BLK, KNN, GW), lambda i: (i, 0, 0)),
            _full((3, C_OUT)), _full((1, C_OUT)),
            _full((3, C_OUT)), _full((1, C_OUT)),
            _full((3, C_OUT)), _full((1, C_OUT)),
            _full((3, C_OUT)), _full((1, C_OUT)),
            _full((C_OUT, C_OUT)), _full((1, C_OUT)),
        ],
        out_specs=[
            pl.BlockSpec((BLK, C_OUT), lambda i: (i, 0)),
            pl.BlockSpec((BLK, C_OUT), lambda i: (i, 0)),
        ],
        out_shape=[
            jax.ShapeDtypeStruct((BN, C_OUT), jnp.float32),
            jax.ShapeDtypeStruct((BN, C_OUT), jnp.float32),
        ],
    )(xyz_flat, gath_flat, W_q, b2(b_q), W_k, b2(b_k), W_v, b2(b_v),
      W_res, b2(b_res), W_ffn, b2(b_ffn))

    out = pl.pallas_call(
        _bn_kernel,
        in_specs=[
            _full((BN, C_OUT)), _full((BN, C_OUT)),
            _full((1, C_OUT)), _full((1, C_OUT)),
            _full((1, C_OUT)), _full((1, C_OUT)),
        ],
        out_specs=_full((BN, C_OUT)),
        out_shape=jax.ShapeDtypeStruct((BN, C_OUT), jnp.float32),
    )(yr, yf, b2(g_res), b2(be_res), b2(g_ffn), b2(be_ffn))

    return out.reshape(B, N, C_OUT)


# speedup vs baseline: 8.6356x; 1.0206x over previous
import functools

import jax
import jax.numpy as jnp
import numpy as np
from jax import lax
from jax.experimental import pallas as pl
from jax.experimental.pallas import tpu as pltpu
from jax.experimental.pallas import tpu_sc as plsc

KNN = 32
C_OUT = 128
BLK = 256
GW = 4
CHUNK = 128
NBUF = 8


def _topk_kernel(x_ref, bt_ref, idx_ref):
    x = x_ref[0]
    bt = bt_ref[0]
    n = bt.shape[1]
    xb = None
    x2 = None
    b2 = None
    for c in range(3):
        xc = x[:, c:c + 1]
        bc = bt[c:c + 1, :]
        xcl = xc.astype(jnp.bfloat16).astype(jnp.float32)
        bcl = bc.astype(jnp.bfloat16).astype(jnp.float32)
        t = xcl * bcl
        xb = t if xb is None else xb + t
        x2 = xc * xc if x2 is None else x2 + xc * xc
        b2 = bc * bc if b2 is None else b2 + bc * bc
    d = ((-2.0 * xb) + x2) + b2
    iota = jax.lax.broadcasted_iota(jnp.int32, d.shape, 1)
    cols = []
    for k in range(KNN):
        rmin = jnp.min(d, axis=1, keepdims=True)
        cand = jnp.where(d == rmin, iota, jnp.int32(n))
        amin = jnp.min(cand, axis=1, keepdims=True)
        cols.append(amin)
        d = jnp.where(iota == amin, jnp.float32(jnp.inf), d)
    base = pl.program_id(0) * n
    idx_ref[0] = jnp.concatenate(cols, axis=1) + base


def _attn_kernel(x_ref, g_ref, wq_ref, bq_ref, wk_ref, bk_ref, wv_ref,
                 bv_ref, wr_ref, br_ref, wf_ref, bf_ref, yr_ref, yf_ref):
    def _r(v):
        return v.astype(jnp.bfloat16).astype(jnp.float32)

    x = x_ref[...]
    g = g_ref[...]
    wq = _r(wq_ref[...])
    wk = _r(wk_ref[...])
    wv = _r(wv_ref[...])
    wr = _r(wr_ref[...])
    lq = None
    yr = None
    lk = None
    lv = None
    for c in range(3):
        xc = x[:, c:c + 1]
        xcl = _r(xc)
        lq = (0.0 if lq is None else lq) + xcl * wq[c:c + 1, :]
        yr = (0.0 if yr is None else yr) + xcl * wr[c:c + 1, :]
        rel_c = _r(g[:, :, c:c + 1] - xc[:, :, None])
        lk = (0.0 if lk is None else lk) + rel_c * wk[c:c + 1, :][None]
        lv = (0.0 if lv is None else lv) + rel_c * wv[c:c + 1, :][None]
    lq = lq + bq_ref[...]
    yr = yr + br_ref[...]
    lk = lk + bk_ref[...][None]
    lv = lv + bv_ref[...][None]
    energy = (lq[:, None, :] - lk) / np.float32(np.sqrt(C_OUT))
    m = jnp.max(energy, axis=1, keepdims=True)
    e = jnp.exp(energy - m)
    p = e / jnp.sum(e, axis=1, keepdims=True)
    a = p - jnp.sum(p, axis=1, keepdims=True)
    ctx = jnp.max(a * lv, axis=1)
    yr_ref[...] = yr
    yf_ref[...] = jnp.dot(ctx.astype(jnp.bfloat16),
                          wf_ref[...].astype(jnp.bfloat16),
                          preferred_element_type=jnp.float32) + bf_ref[...]


def _make_sc_gather(n_idx, n_rows):
    info = plsc.get_sparse_core_info()
    nw = info.num_cores * info.num_subcores
    n_chunks = n_idx // CHUNK
    cpw = n_chunks // nw
    assert cpw % NBUF == 0 and n_chunks % nw == 0
    mesh = plsc.VectorSubcoreMesh(core_axis_name="c", subcore_axis_name="s")

    bufs = [pltpu.VMEM((CHUNK, GW), jnp.float32) for _ in range(NBUF)]
    gsems = [pltpu.SemaphoreType.DMA for _ in range(NBUF)]
    ssems = [pltpu.SemaphoreType.DMA for _ in range(NBUF)]

    @functools.partial(
        pl.kernel,
        mesh=mesh,
        compiler_params=pltpu.CompilerParams(use_tc_tiling_on_sc=False),
        out_type=jax.ShapeDtypeStruct((n_idx, GW), jnp.float32),
        scratch_types=[pltpu.VMEM((cpw, CHUNK), jnp.int32)] + bufs + gsems
        + ssems,
    )
    def gather(idx_hbm, table_hbm, out_hbm, idx_v, *bs):
        buf = bs[:NBUF]
        gsem = bs[NBUF:2 * NBUF]
        ssem = bs[2 * NBUF:]
        wid = lax.axis_index("s") * info.num_cores + lax.axis_index("c")
        c0 = wid * cpw
        pltpu.sync_copy(idx_hbm.at[pl.ds(c0, cpw)], idx_v)

        def gather_cp(c, b):
            return pltpu.make_async_copy(table_hbm.at[idx_v.at[c]], buf[b],
                                         gsem[b])

        def store_cp(c, b):
            return pltpu.make_async_copy(
                buf[b], out_hbm.at[pl.ds((c0 + c) * CHUNK, CHUNK)], ssem[b])

        for b in range(NBUF):
            gather_cp(b, b).start()

        def body(t, _):
            c = t * NBUF
            for b in range(NBUF):
                gather_cp(c + b, b).wait()
                store_cp(c + b, b).start()
            for b in range(NBUF):
                store_cp(c + b, b).wait()

                @pl.when(c + b + NBUF < cpw)
                def _():
                    gather_cp(c + b + NBUF, b).start()
            return 0

        lax.fori_loop(0, cpw // NBUF, body, 0)

    return gather


def _leaky(x):
    return jnp.where(x >= 0, x, 0.2 * x)


def _bn_kernel(yr_ref, yf_ref, gr_ref, ber_ref, gf_ref, bef_ref, out_ref):
    def bn(y, gamma, beta):
        mean = jnp.mean(y, axis=0, keepdims=True)
        var = jnp.mean((y - mean) ** 2, axis=0, keepdims=True)
        return gamma * (y - mean) / jnp.sqrt(var + 1e-5) + beta

    hr = _leaky(bn(yr_ref[...], gr_ref[...], ber_ref[...]))
    hf = _leaky(bn(yf_ref[...], gf_ref[...], bef_ref[...]))
    out_ref[...] = hr + hf


def _full(shape):
    nd = len(shape)
    return pl.BlockSpec(shape, lambda *args: (0,) * nd)


def kernel(xyz, base_xyz, W_q, b_q, W_k, b_k, W_v, b_v, W_res, b_res,
           g_res, be_res, W_ffn, b_ffn, g_ffn, be_ffn):
    B, N, _ = xyz.shape
    BN = B * N
    base_t = jnp.transpose(base_xyz, (0, 2, 1))

    idx = pl.pallas_call(
        _topk_kernel,
        grid=(B, N // BLK),
        in_specs=[
            pl.BlockSpec((1, BLK, 3), lambda b, i: (b, i, 0)),
            pl.BlockSpec((1, 3, N), lambda b, i: (b, 0, 0)),
        ],
        out_specs=pl.BlockSpec((1, BLK, KNN), lambda b, i: (b, i, 0)),
        out_shape=jax.ShapeDtypeStruct((B, N, KNN), jnp.int32),
    )(xyz, base_t)

    n_idx = BN * KNN
    table = jnp.pad(xyz.reshape(BN, 3), ((0, 0), (0, GW - 3)))
    idx_chunks = idx.reshape(n_idx // CHUNK, CHUNK)
    gath_flat = _make_sc_gather(n_idx, BN)(idx_chunks, table)
    gath_flat = gath_flat.reshape(BN, KNN, GW)

    xyz_flat = xyz.reshape(BN, 3)
    b2 = lambda v: v.reshape(1, C_OUT)
    yr, yf = pl.pallas_call(
        _attn_kernel,
        grid=(BN // BLK,),
        in_specs=[
            pl.BlockSpec((BLK, 3), lambda i: (i, 0)),
            pl.BlockSpec((BLK, KNN, GW), lambda i: (i, 0, 0)),
            _full((3, C_OUT)), _full((1, C_OUT)),
            _full((3, C_OUT)), _full((1, C_OUT)),
            _full((3, C_OUT)), _full((1, C_OUT)),
            _full((3, C_OUT)), _full((1, C_OUT)),
            _full((C_OUT, C_OUT)), _full((1, C_OUT)),
        ],
        out_specs=[
            pl.BlockSpec((BLK, C_OUT), lambda i: (i, 0)),
            pl.BlockSpec((BLK, C_OUT), lambda i: (i, 0)),
        ],
        out_shape=[
            jax.ShapeDtypeStruct((BN, C_OUT), jnp.float32),
            jax.ShapeDtypeStruct((BN, C_OUT), jnp.float32),
        ],
    )(xyz_flat, gath_flat, W_q, b2(b_q), W_k, b2(b_k), W_v, b2(b_v),
      W_res, b2(b_res), W_ffn, b2(b_ffn))

    out = pl.pallas_call(
        _bn_kernel,
        in_specs=[
            _full((BN, C_OUT)), _full((BN, C_OUT)),
            _full((1, C_OUT)), _full((1, C_OUT)),
            _full((1, C_OUT)), _full((1, C_OUT)),
        ],
        out_specs=_full((BN, C_OUT)),
        out_shape=jax.ShapeDtypeStruct((BN, C_OUT), jnp.float32),
    )(yr, yf, b2(g_res), b2(be_res), b2(g_ffn), b2(be_ffn))

    return out.reshape(B, N, C_OUT)
